# Initial kernel scaffold; baseline (speedup 1.0000x reference)
import jax
import jax.numpy as jnp
from jax.experimental import pallas as pl


def kernel(src, dst, t, neg_dst, memory, emb_W, emb_b, fc1_W, fc1_b, fc2_W, fc2_b, msg_W1, msg_b1, msg_W2, msg_b2, gru_W_ih, gru_W_hh, gru_b_ih, gru_b_hh):
    raise NotImplementedError("write your pallas kernel here")


# speedup vs baseline: 9.9900x; 9.9900x over previous
import functools

import jax
import jax.numpy as jnp
from jax import lax
from jax.experimental import pallas as pl
from jax.experimental.pallas import tpu as pltpu
from jax.experimental.pallas import tpu_sc as plsc

_NC = 2
_NS = 16
_NW = _NC * _NS
_CH = 128


def _wid():
  return lax.axis_index("s") * _NC + lax.axis_index("c")


def _sc_mesh():
  return plsc.VectorSubcoreMesh(core_axis_name="c", subcore_axis_name="s",
                                num_cores=_NC, num_subcores=_NS)


def _sc_scratch(D):
  return [
      pltpu.VMEM((_CH,), jnp.int32),
      pltpu.VMEM((_CH, D), jnp.float32),
      pltpu.SemaphoreType.DMA,
  ]


def _make_gather3(N, B, D):
  bpw = B // _NW
  nch = bpw // _CH

  @functools.partial(
      pl.kernel,
      out_type=tuple(jax.ShapeDtypeStruct((B, D), jnp.float32) for _ in range(3)),
      mesh=_sc_mesh(),
      scratch_types=_sc_scratch(D),
  )
  def gather3(mem, src, dst, neg, o_s, o_d, o_n, idx_v, rows_v, sem):
    base = _wid() * bpw
    for ih, oh in ((src, o_s), (dst, o_d), (neg, o_n)):
      for j in range(nch):
        off = base + j * _CH
        pltpu.sync_copy(ih.at[pl.ds(off, _CH)], idx_v)
        pltpu.async_copy(mem.at[idx_v], rows_v, sem).wait()
        pltpu.sync_copy(rows_v, oh.at[pl.ds(off, _CH)])

  return gather3


def _make_gather1(N, B, D):
  bpw = B // _NW
  nch = bpw // _CH

  @functools.partial(
      pl.kernel,
      out_type=jax.ShapeDtypeStruct((B, D), jnp.float32),
      mesh=_sc_mesh(),
      scratch_types=_sc_scratch(D),
  )
  def gather1(table, idx, out, idx_v, rows_v, sem):
    base = _wid() * bpw
    for j in range(nch):
      off = base + j * _CH
      pltpu.sync_copy(idx.at[pl.ds(off, _CH)], idx_v)
      pltpu.async_copy(table.at[idx_v], rows_v, sem).wait()
      pltpu.sync_copy(rows_v, out.at[pl.ds(off, _CH)])

  return gather1


def _make_scatter(N, B, D):
  bpw = B // _NW
  nch = bpw // _CH

  @functools.partial(
      pl.kernel,
      out_type=(),
      mesh=_sc_mesh(),
      scratch_types=_sc_scratch(D),
  )
  def scatter(table, idx, upd, idx_v, rows_v, sem):
    base = _wid() * bpw
    for j in range(nch):
      off = base + j * _CH
      pltpu.sync_copy(idx.at[pl.ds(off, _CH)], idx_v)
      pltpu.sync_copy(upd.at[pl.ds(off, _CH)], rows_v)
      pltpu.async_copy(rows_v, table.at[idx_v], sem).wait()

  return scatter


def _gru(x, h, wih, whh, bih, bhh, D):
  dot = lambda p, q: jnp.dot(p, q, preferred_element_type=jnp.float32)
  gx = dot(x, wih) + bih
  gh = dot(h, whh) + bhh
  r = jax.nn.sigmoid(gx[:, :D] + gh[:, :D])
  z = jax.nn.sigmoid(gx[:, D:2 * D] + gh[:, D:2 * D])
  n = jnp.tanh(gx[:, 2 * D:] + r * gh[:, 2 * D:])
  return (1.0 - z) * n + z * h


def _make_dense(B, D, bs):
  grid = B // bs
  row = lambda: pl.BlockSpec((bs, D), lambda i: (i, 0))
  full = lambda r, c: pl.BlockSpec((r, c), lambda i: (0, 0))

  def body(ms, md, mn, embw, embb, f1a, f1b, f1b_, f2r, f2b, w1a, w1b, mb1,
           w2, mb2, wih, whh, bih, bhh, pos_o, neg_o, upd_o, msd_o):
    dot = lambda p, q: jnp.dot(p, q, preferred_element_type=jnp.float32)
    a = ms[...]
    b = md[...]
    c = mn[...]
    za = dot(a, embw[...]) + embb[...]
    zb = dot(b, embw[...]) + embb[...]
    zc = dot(c, embw[...]) + embb[...]
    t1 = dot(za, f1a[...])
    hp = jnp.maximum(t1 + dot(zb, f1b[...]) + f1b_[...], 0.0)
    hn = jnp.maximum(t1 + dot(zc, f1b[...]) + f1b_[...], 0.0)
    pos_o[...] = jnp.sum(hp * f2r[...], axis=1, keepdims=True) + f2b[...]
    neg_o[...] = jnp.sum(hn * f2r[...], axis=1, keepdims=True) + f2b[...]
    ab = dot(a, w1a[...])
    ba = dot(b, w1a[...])
    bb = dot(b, w1b[...])
    aa = dot(a, w1b[...])
    msg_s = dot(jnp.maximum(ab + bb + mb1[...], 0.0), w2[...]) + mb2[...]
    msg_d = dot(jnp.maximum(ba + aa + mb1[...], 0.0), w2[...]) + mb2[...]
    msd_o[...] = msg_d
    upd_o[...] = _gru(msg_s, a, wih[...], whh[...], bih[...], bhh[...], D)

  return pl.pallas_call(
      body,
      grid=(grid,),
      in_specs=[
          row(), row(), row(),
          full(D, D), full(1, D),
          full(D, D), full(D, D), full(1, D),
          full(1, D), full(1, 1),
          full(D, D), full(D, D), full(1, D),
          full(D, D), full(1, D),
          full(D, 3 * D), full(D, 3 * D), full(1, 3 * D), full(1, 3 * D),
      ],
      out_specs=[
          pl.BlockSpec((bs, 1), lambda i: (i, 0)),
          pl.BlockSpec((bs, 1), lambda i: (i, 0)),
          row(), row(),
      ],
      out_shape=[
          jax.ShapeDtypeStruct((B, 1), jnp.float32),
          jax.ShapeDtypeStruct((B, 1), jnp.float32),
          jax.ShapeDtypeStruct((B, D), jnp.float32),
          jax.ShapeDtypeStruct((B, D), jnp.float32),
      ],
  )


def _make_gru_dst(B, D, bs):
  grid = B // bs
  row = lambda: pl.BlockSpec((bs, D), lambda i: (i, 0))
  full = lambda r, c: pl.BlockSpec((r, c), lambda i: (0, 0))

  def body(msd, cur, wih, whh, bih, bhh, out):
    out[...] = _gru(msd[...], cur[...], wih[...], whh[...], bih[...],
                    bhh[...], D)

  return pl.pallas_call(
      body,
      grid=(grid,),
      in_specs=[row(), row(), full(D, 3 * D), full(D, 3 * D),
                full(1, 3 * D), full(1, 3 * D)],
      out_specs=row(),
      out_shape=jax.ShapeDtypeStruct((B, D), jnp.float32),
  )


def kernel(src, dst, t, neg_dst, memory, emb_W, emb_b, fc1_W, fc1_b, fc2_W,
           fc2_b, msg_W1, msg_b1, msg_W2, msg_b2, gru_W_ih, gru_W_hh,
           gru_b_ih, gru_b_hh):
  B = src.shape[0]
  N, D = memory.shape
  src = src.astype(jnp.int32)
  dst = dst.astype(jnp.int32)
  neg_dst = neg_dst.astype(jnp.int32)

  embb = emb_b.reshape(1, D)
  f1a, f1b = fc1_W[:D], fc1_W[D:]
  f1bias = fc1_b.reshape(1, D)
  f2r = fc2_W.reshape(1, D)
  f2b = fc2_b.reshape(1, 1)
  w1a, w1b = msg_W1[:D], msg_W1[D:2 * D]
  mb1 = msg_b1.reshape(1, D)
  mb2 = msg_b2.reshape(1, D)
  bih = gru_b_ih.reshape(1, 3 * D)
  bhh = gru_b_hh.reshape(1, 3 * D)

  mem_src, mem_dst, mem_neg = _make_gather3(N, B, D)(memory, src, dst, neg_dst)

  pos, neg, upd_src, msg_d = _make_dense(B, D, 2048)(
      mem_src, mem_dst, mem_neg, emb_W, embb, f1a, f1b, f1bias, f2r, f2b,
      w1a, w1b, mb1, msg_W2, mb2, gru_W_ih, gru_W_hh, bih, bhh)

  out_ref = jax.new_ref(memory)
  _make_scatter(N, B, D)(out_ref, src, upd_src)
  cur_dst = _make_gather1(N, B, D)(out_ref, dst)
  upd_dst = _make_gru_dst(B, D, 2048)(msg_d, cur_dst, gru_W_ih, gru_W_hh,
                                      bih, bhh)
  _make_scatter(N, B, D)(out_ref, dst, upd_dst)
  return pos, neg, out_ref[...]


# speedup vs baseline: 10.0042x; 1.0014x over previous
import functools

import jax
import jax.numpy as jnp
from jax import lax
from jax.experimental import pallas as pl
from jax.experimental.pallas import tpu as pltpu
from jax.experimental.pallas import tpu_sc as plsc

_NC = 2
_NS = 16
_NW = _NC * _NS
_CH = 128


def _wid():
  return lax.axis_index("s") * _NC + lax.axis_index("c")


def _sc_mesh():
  return plsc.VectorSubcoreMesh(core_axis_name="c", subcore_axis_name="s",
                                num_cores=_NC, num_subcores=_NS)


def _sc_scratch(D):
  return [
      pltpu.VMEM((_CH,), jnp.int32),
      pltpu.VMEM((_CH, D), jnp.float32),
      pltpu.SemaphoreType.DMA,
  ]


def _make_gather3(N, B, D):
  bpw = B // _NW
  nch = bpw // _CH

  @functools.partial(
      pl.kernel,
      out_type=tuple(jax.ShapeDtypeStruct((B, D), jnp.float32) for _ in range(3)),
      mesh=_sc_mesh(),
      scratch_types=_sc_scratch(D),
  )
  def gather3(mem, src, dst, neg, o_s, o_d, o_n, idx_v, rows_v, sem):
    base = _wid() * bpw
    for ih, oh in ((src, o_s), (dst, o_d), (neg, o_n)):
      for j in range(nch):
        off = base + j * _CH
        pltpu.sync_copy(ih.at[pl.ds(off, _CH)], idx_v)
        pltpu.async_copy(mem.at[idx_v], rows_v, sem).wait()
        pltpu.sync_copy(rows_v, oh.at[pl.ds(off, _CH)])

  return gather3


def _make_gather1(N, B, D):
  bpw = B // _NW
  nch = bpw // _CH

  @functools.partial(
      pl.kernel,
      out_type=jax.ShapeDtypeStruct((B, D), jnp.float32),
      mesh=_sc_mesh(),
      scratch_types=_sc_scratch(D),
  )
  def gather1(table, idx, out, idx_v, rows_v, sem):
    base = _wid() * bpw
    for j in range(nch):
      off = base + j * _CH
      pltpu.sync_copy(idx.at[pl.ds(off, _CH)], idx_v)
      pltpu.async_copy(table.at[idx_v], rows_v, sem).wait()
      pltpu.sync_copy(rows_v, out.at[pl.ds(off, _CH)])

  return gather1


def _make_scatter(N, B, D):
  bpw = B // _NW
  nch = bpw // _CH

  @functools.partial(
      pl.kernel,
      out_type=(),
      mesh=_sc_mesh(),
      scratch_types=_sc_scratch(D),
  )
  def scatter(table, idx, upd, idx_v, rows_v, sem):
    base = _wid() * bpw
    for j in range(nch):
      off = base + j * _CH
      pltpu.sync_copy(idx.at[pl.ds(off, _CH)], idx_v)
      pltpu.sync_copy(upd.at[pl.ds(off, _CH)], rows_v)
      pltpu.async_copy(rows_v, table.at[idx_v], sem).wait()

  return scatter


def _gru(x, h, wih, whh, bih, bhh, D):
  dot = lambda p, q: jnp.dot(p, q, preferred_element_type=jnp.float32)
  gx = dot(x, wih) + bih
  gh = dot(h, whh) + bhh
  r = jax.nn.sigmoid(gx[:, :D] + gh[:, :D])
  z = jax.nn.sigmoid(gx[:, D:2 * D] + gh[:, D:2 * D])
  n = jnp.tanh(gx[:, 2 * D:] + r * gh[:, 2 * D:])
  return (1.0 - z) * n + z * h


def _make_dense(B, D, bs):
  grid = B // bs
  row = lambda: pl.BlockSpec((bs, D), lambda i: (i, 0))
  full = lambda r, c: pl.BlockSpec((r, c), lambda i: (0, 0))

  def body(ms, md, mn, embw, embb, f1a, f1b, f1b_, f2r, f2b, w1a, w1b, mb1,
           w2, mb2, wih, whh, bih, bhh, pos_o, neg_o, upd_o, msd_o):
    dot = lambda p, q: jnp.dot(p, q, preferred_element_type=jnp.float32)
    a = ms[...]
    b = md[...]
    c = mn[...]
    za = dot(a, embw[...]) + embb[...]
    zb = dot(b, embw[...]) + embb[...]
    zc = dot(c, embw[...]) + embb[...]
    t1 = dot(za, f1a[...])
    hp = jnp.maximum(t1 + dot(zb, f1b[...]) + f1b_[...], 0.0)
    hn = jnp.maximum(t1 + dot(zc, f1b[...]) + f1b_[...], 0.0)
    pos_o[...] = jnp.sum(hp * f2r[...], axis=1, keepdims=True) + f2b[...]
    neg_o[...] = jnp.sum(hn * f2r[...], axis=1, keepdims=True) + f2b[...]
    ab = dot(a, w1a[...])
    ba = dot(b, w1a[...])
    bb = dot(b, w1b[...])
    aa = dot(a, w1b[...])
    msg_s = dot(jnp.maximum(ab + bb + mb1[...], 0.0), w2[...]) + mb2[...]
    msg_d = dot(jnp.maximum(ba + aa + mb1[...], 0.0), w2[...]) + mb2[...]
    msd_o[...] = msg_d
    upd_o[...] = _gru(msg_s, a, wih[...], whh[...], bih[...], bhh[...], D)

  return pl.pallas_call(
      body,
      grid=(grid,),
      in_specs=[
          row(), row(), row(),
          full(D, D), full(1, D),
          full(D, D), full(D, D), full(1, D),
          full(1, D), full(1, 1),
          full(D, D), full(D, D), full(1, D),
          full(D, D), full(1, D),
          full(D, 3 * D), full(D, 3 * D), full(1, 3 * D), full(1, 3 * D),
      ],
      out_specs=[
          pl.BlockSpec((bs, 1), lambda i: (i, 0)),
          pl.BlockSpec((bs, 1), lambda i: (i, 0)),
          row(), row(),
      ],
      out_shape=[
          jax.ShapeDtypeStruct((B, 1), jnp.float32),
          jax.ShapeDtypeStruct((B, 1), jnp.float32),
          jax.ShapeDtypeStruct((B, D), jnp.float32),
          jax.ShapeDtypeStruct((B, D), jnp.float32),
      ],
  )


def _make_gru_dst(B, D, bs):
  grid = B // bs
  row = lambda: pl.BlockSpec((bs, D), lambda i: (i, 0))
  full = lambda r, c: pl.BlockSpec((r, c), lambda i: (0, 0))

  def body(msd, cur, wih, whh, bih, bhh, out):
    out[...] = _gru(msd[...], cur[...], wih[...], whh[...], bih[...],
                    bhh[...], D)

  return pl.pallas_call(
      body,
      grid=(grid,),
      in_specs=[row(), row(), full(D, 3 * D), full(D, 3 * D),
                full(1, 3 * D), full(1, 3 * D)],
      out_specs=row(),
      out_shape=jax.ShapeDtypeStruct((B, D), jnp.float32),
  )


def kernel(src, dst, t, neg_dst, memory, emb_W, emb_b, fc1_W, fc1_b, fc2_W,
           fc2_b, msg_W1, msg_b1, msg_W2, msg_b2, gru_W_ih, gru_W_hh,
           gru_b_ih, gru_b_hh):
  B = src.shape[0]
  N, D = memory.shape
  src = src.astype(jnp.int32)
  dst = dst.astype(jnp.int32)
  neg_dst = neg_dst.astype(jnp.int32)

  embb = emb_b.reshape(1, D)
  f1a, f1b = fc1_W[:D], fc1_W[D:]
  f1bias = fc1_b.reshape(1, D)
  f2r = fc2_W.reshape(1, D)
  f2b = fc2_b.reshape(1, 1)
  w1a, w1b = msg_W1[:D], msg_W1[D:2 * D]
  mb1 = msg_b1.reshape(1, D)
  mb2 = msg_b2.reshape(1, D)
  bih = gru_b_ih.reshape(1, 3 * D)
  bhh = gru_b_hh.reshape(1, 3 * D)

  out_ref = jax.new_ref(memory)
  mem_src, mem_dst, mem_neg = _make_gather3(N, B, D)(memory, src, dst, neg_dst)

  pos, neg, upd_src, msg_d = _make_dense(B, D, 2048)(
      mem_src, mem_dst, mem_neg, emb_W, embb, f1a, f1b, f1bias, f2r, f2b,
      w1a, w1b, mb1, msg_W2, mb2, gru_W_ih, gru_W_hh, bih, bhh)

  _make_scatter(N, B, D)(out_ref, src, upd_src)
  cur_dst = _make_gather1(N, B, D)(out_ref, dst)
  upd_dst = _make_gru_dst(B, D, 2048)(msg_d, cur_dst, gru_W_ih, gru_W_hh,
                                      bih, bhh)
  _make_scatter(N, B, D)(out_ref, dst, upd_dst)
  return pos, neg, out_ref[...]
